# baseline (speedup 1.0000x reference)
import functools
import math

import jax
import jax.numpy as jnp
from jax import lax
from jax.experimental import pallas as pl
from jax.experimental.pallas import tpu as pltpu
from jax.experimental.pallas import tpu_sc as plsc

_N = 10000
_E = 320000
_D = 128
_SH = 9
_EA = 16
_INV_SQRT_DEG = 1.0 / math.sqrt(32.0)

_NC = 2
_NS = 16
_NW = _NC * _NS
_EPW = _E // _NW
_G = 80
_NCHUNK = _EPW // _G
_NP = 10240
_RPS = _NP // _NS
_ZR = 128

_BE = 2000
_BN = 2000


def _i0(i):
    return i * 0


_BR = 1000


def _coef_body(ea_ref, sh_ref, w_ref, o_ref):
    x = jnp.concatenate([ea_ref[...], sh_ref[...]], axis=1)
    y = lax.dot_general(
        x, w_ref[...], (((1,), (0,)), ((), ())),
        preferred_element_type=jnp.float32)
    c = jax.nn.sigmoid(y[:, :1024]) * y[:, 1024:]
    o_ref[...] = c.reshape(8 * _BR, _D)


def _compute_coef(ea128, shp128, W_cat2):
    return pl.pallas_call(
        _coef_body,
        grid=(_E // (8 * _BR),),
        in_specs=[
            pl.BlockSpec((_BR, _D), lambda i: (i, _i0(i))),
            pl.BlockSpec((_BR, _D), lambda i: (i, _i0(i))),
            pl.BlockSpec((2 * _D, 8 * 2 * _D), lambda i: (_i0(i), _i0(i))),
        ],
        out_specs=pl.BlockSpec((8 * _BR, _D), lambda i: (i, _i0(i))),
        out_shape=jax.ShapeDtypeStruct((_E, _D), jnp.float32),
    )(ea128, shp128, W_cat2)


def _loop32(lo, hi, body_fn):
    lax.fori_loop(jnp.int32(lo), jnp.int32(hi),
                  lambda i, c: (body_fn(i), c)[1], None)


def _sc_body(feat_hbm, eidx_hbm, coef_hbm, out_hbm,
             sidx, didx, rows, coefb, stage, agg_sh, sem):
    cid = lax.axis_index("c").astype(jnp.int32)
    sid = lax.axis_index("s").astype(jnp.int32)
    wid = cid * jnp.int32(_NS) + sid

    def _zero_stage(i):
        def _zs(j16):
            stage.at[i, pl.ds(j16 * jnp.int32(16), 16)][...] = (
                jnp.zeros((16,), jnp.float32))
        _loop32(0, _D // 16, _zs)
    _loop32(0, _ZR, _zero_stage)

    for r in range(0, _RPS, _ZR):
        pltpu.sync_copy(stage, agg_sh.at[pl.ds(sid * jnp.int32(_RPS) + jnp.int32(r), _ZR)])
    plsc.subcore_barrier()

    def _chunk(k):
        off = wid * jnp.int32(_EPW) + k * jnp.int32(_G)
        pltpu.sync_copy(eidx_hbm.at[pl.ds(off, _G)], sidx)
        pltpu.sync_copy(eidx_hbm.at[pl.ds(off + jnp.int32(_E), _G)], didx)
        pltpu.async_copy(feat_hbm.at[sidx], rows, sem).wait()
        pltpu.sync_copy(coef_hbm.at[pl.ds(off, _G)], coefb)

        def _row(i):
            def _mul(j16):
                j = j16 * jnp.int32(16)
                rows.at[i, pl.ds(j, 16)][...] = (
                    rows.at[i, pl.ds(j, 16)][...]
                    * coefb.at[i, pl.ds(j, 16)][...])
            _loop32(0, _D // 16, _mul)
        _loop32(0, _G, _row)

        pltpu.sync_copy(rows, agg_sh.at[didx], add=True)
    _loop32(0, _NCHUNK, _chunk)

    plsc.subcore_barrier()

    for r in range(0, _RPS, _ZR):
        row0 = sid * jnp.int32(_RPS) + jnp.int32(r)
        pltpu.sync_copy(agg_sh.at[pl.ds(row0, _ZR)], stage)
        pltpu.sync_copy(stage, out_hbm.at[cid, pl.ds(row0, _ZR)])


def _sc_scatter(features, eidx, coef):
    mesh = plsc.VectorSubcoreMesh(core_axis_name="c", subcore_axis_name="s")
    k = pl.kernel(
        _sc_body,
        mesh=mesh,
        out_type=jax.ShapeDtypeStruct((_NC, _NP, _D), jnp.float32),
        scratch_types=[
            pltpu.VMEM((_G,), jnp.int32),
            pltpu.VMEM((_G,), jnp.int32),
            pltpu.VMEM((_G, _D), jnp.float32),
            pltpu.VMEM((_G, _D), jnp.float32),
            pltpu.VMEM((_ZR, _D), jnp.float32),
            pltpu.VMEM_SHARED((_NP, _D), jnp.float32),
            pltpu.SemaphoreType.DMA,
        ],
    )
    return k(features, eidx, coef)


def _out_body(agg_ref, wout_ref, cn_ref, ws_ref, o_ref):
    agg = (agg_ref[0] + agg_ref[1]) * _INV_SQRT_DEG
    out = lax.dot_general(
        agg, wout_ref[...], (((1,), (0,)), ((), ())),
        precision=lax.Precision.HIGHEST, preferred_element_type=jnp.float32)
    scale = 1.0 + jnp.tanh(cn_ref[...]) * ws_ref[...]
    o_ref[...] = out * scale


def _epilogue(partials, W_out, c_noise, w_scale):
    return pl.pallas_call(
        _out_body,
        grid=(_N // _BN,),
        in_specs=[
            pl.BlockSpec((_NC, _BN, _D), lambda i: (_i0(i), i, _i0(i))),
            pl.BlockSpec((_D, _D), lambda i: (_i0(i), _i0(i))),
            pl.BlockSpec((_BN, 1), lambda i: (i, _i0(i))),
            pl.BlockSpec((1, _D), lambda i: (_i0(i), _i0(i))),
        ],
        out_specs=pl.BlockSpec((_BN, _D), lambda i: (i, _i0(i))),
        out_shape=jax.ShapeDtypeStruct((_N, _D), jnp.float32),
    )(partials, W_out, c_noise, w_scale)


def kernel(features, coords, edge_index, edge_attr, edge_sh, c_noise,
           W_attr, W_sh, W_out, w_scale):
    out_dtype = jnp.result_type(features.dtype, W_attr.dtype, W_out.dtype)
    eidx = edge_index.reshape(-1).astype(jnp.int32)
    ea128 = edge_attr.reshape(_E // 8, _D)
    shp128 = jnp.pad(edge_sh, ((0, 0), (0, _EA - _SH))).reshape(_E // 8, _D)
    Wg = jnp.zeros((_D, 8 * _D), jnp.float32)
    Wsd = jnp.zeros((_D, 8 * _D), jnp.float32)
    Wa32 = W_attr.astype(jnp.float32)
    Wsp = jnp.pad(W_sh.astype(jnp.float32), ((0, _EA - _SH), (0, 0)))
    for t in range(8):
        Wg = Wg.at[_EA * t:_EA * (t + 1), _D * t:_D * (t + 1)].set(Wa32)
        Wsd = Wsd.at[_EA * t:_EA * (t + 1), _D * t:_D * (t + 1)].set(Wsp)
    W_cat2 = jnp.zeros((2 * _D, 16 * _D), jnp.float32)
    W_cat2 = W_cat2.at[:_D, :8 * _D].set(Wg)
    W_cat2 = W_cat2.at[_D:, 8 * _D:].set(Wsd)
    coef = _compute_coef(ea128, shp128, W_cat2)
    partials = _sc_scatter(features, eidx, coef)
    new_features = _epilogue(
        partials, W_out.astype(jnp.float32),
        c_noise.reshape(_N, 1).astype(jnp.float32),
        w_scale.reshape(1, _D).astype(jnp.float32))
    return (coords, new_features.astype(out_dtype))


# speedup vs baseline: 1.4780x; 1.4780x over previous
import functools
import math

import jax
import jax.numpy as jnp
from jax import lax
from jax.experimental import pallas as pl
from jax.experimental.pallas import tpu as pltpu
from jax.experimental.pallas import tpu_sc as plsc

_N = 10000
_E = 320000
_D = 128
_SH = 9
_EA = 16
_INV_SQRT_DEG = 1.0 / math.sqrt(32.0)

_NC = 2
_NS = 16
_NW = _NC * _NS
_EPW = _E // _NW
_G = 80
_NCHUNK = _EPW // _G
_NP = 10240
_RPS = _NP // _NS
_ZR = 128

_BE = 6400
_BN = 2000


def _i0(i):
    return i * 0


def _coef_body(eat_ref, sht_ref, wa_ref, ws_ref, o_ref):
    g = lax.dot_general(
        eat_ref[...], wa_ref[...], (((0,), (0,)), ((), ())),
        preferred_element_type=jnp.float32)
    s = lax.dot_general(
        sht_ref[...], ws_ref[...], (((0,), (0,)), ((), ())),
        preferred_element_type=jnp.float32)
    o_ref[...] = jax.nn.sigmoid(g) * s


def _compute_coef(eaT, shT, Wa, Ws):
    return pl.pallas_call(
        _coef_body,
        grid=(_E // _BE,),
        in_specs=[
            pl.BlockSpec((_EA, _BE), lambda i: (_i0(i), i)),
            pl.BlockSpec((_SH, _BE), lambda i: (_i0(i), i)),
            pl.BlockSpec((_EA, _D), lambda i: (_i0(i), _i0(i))),
            pl.BlockSpec((_SH, _D), lambda i: (_i0(i), _i0(i))),
        ],
        out_specs=pl.BlockSpec((_BE, _D), lambda i: (i, _i0(i))),
        out_shape=jax.ShapeDtypeStruct((_E, _D), jnp.float32),
    )(eaT, shT, Wa, Ws)


def _loop32(lo, hi, body_fn):
    lax.fori_loop(jnp.int32(lo), jnp.int32(hi),
                  lambda i, c: (body_fn(i), c)[1], None)


def _sc_body(feat_hbm, eidx_hbm, coef_hbm, out_hbm,
             sidx, didx, rows, coefb, stage, agg_sh, sem):
    cid = lax.axis_index("c").astype(jnp.int32)
    sid = lax.axis_index("s").astype(jnp.int32)
    wid = cid * jnp.int32(_NS) + sid

    def _zero_stage(i):
        def _zs(j16):
            stage.at[i, pl.ds(j16 * jnp.int32(16), 16)][...] = (
                jnp.zeros((16,), jnp.float32))
        _loop32(0, _D // 16, _zs)
    _loop32(0, _ZR, _zero_stage)

    for r in range(0, _RPS, _ZR):
        pltpu.sync_copy(stage, agg_sh.at[pl.ds(sid * jnp.int32(_RPS) + jnp.int32(r), _ZR)])
    plsc.subcore_barrier()

    def _chunk(k):
        off = wid * jnp.int32(_EPW) + k * jnp.int32(_G)
        pltpu.sync_copy(eidx_hbm.at[pl.ds(off, _G)], sidx)
        pltpu.sync_copy(eidx_hbm.at[pl.ds(off + jnp.int32(_E), _G)], didx)
        pltpu.async_copy(feat_hbm.at[sidx], rows, sem).wait()
        pltpu.sync_copy(coef_hbm.at[pl.ds(off, _G)], coefb)

        def _row(i):
            def _mul(j16):
                j = j16 * jnp.int32(16)
                rows.at[i, pl.ds(j, 16)][...] = (
                    rows.at[i, pl.ds(j, 16)][...]
                    * coefb.at[i, pl.ds(j, 16)][...])
            _loop32(0, _D // 16, _mul)
        _loop32(0, _G, _row)

        pltpu.sync_copy(rows, agg_sh.at[didx], add=True)
    _loop32(0, _NCHUNK, _chunk)

    plsc.subcore_barrier()

    for r in range(0, _RPS, _ZR):
        row0 = sid * jnp.int32(_RPS) + jnp.int32(r)
        pltpu.sync_copy(agg_sh.at[pl.ds(row0, _ZR)], stage)
        pltpu.sync_copy(stage, out_hbm.at[cid, pl.ds(row0, _ZR)])


def _sc_scatter(features, eidx, coef):
    mesh = plsc.VectorSubcoreMesh(core_axis_name="c", subcore_axis_name="s")
    k = pl.kernel(
        _sc_body,
        mesh=mesh,
        out_type=jax.ShapeDtypeStruct((_NC, _NP, _D), jnp.float32),
        scratch_types=[
            pltpu.VMEM((_G,), jnp.int32),
            pltpu.VMEM((_G,), jnp.int32),
            pltpu.VMEM((_G, _D), jnp.float32),
            pltpu.VMEM((_G, _D), jnp.float32),
            pltpu.VMEM((_ZR, _D), jnp.float32),
            pltpu.VMEM_SHARED((_NP, _D), jnp.float32),
            pltpu.SemaphoreType.DMA,
        ],
    )
    return k(features, eidx, coef)


def _out_body(agg_ref, wout_ref, cn_ref, ws_ref, o_ref):
    agg = (agg_ref[0] + agg_ref[1]) * _INV_SQRT_DEG
    out = lax.dot_general(
        agg, wout_ref[...], (((1,), (0,)), ((), ())),
        precision=lax.Precision.HIGHEST, preferred_element_type=jnp.float32)
    scale = 1.0 + jnp.tanh(cn_ref[...]) * ws_ref[...]
    o_ref[...] = out * scale


def _epilogue(partials, W_out, c_noise, w_scale):
    return pl.pallas_call(
        _out_body,
        grid=(_N // _BN,),
        in_specs=[
            pl.BlockSpec((_NC, _BN, _D), lambda i: (_i0(i), i, _i0(i))),
            pl.BlockSpec((_D, _D), lambda i: (_i0(i), _i0(i))),
            pl.BlockSpec((_BN, 1), lambda i: (i, _i0(i))),
            pl.BlockSpec((1, _D), lambda i: (_i0(i), _i0(i))),
        ],
        out_specs=pl.BlockSpec((_BN, _D), lambda i: (i, _i0(i))),
        out_shape=jax.ShapeDtypeStruct((_N, _D), jnp.float32),
    )(partials, W_out, c_noise, w_scale)


def kernel(features, coords, edge_index, edge_attr, edge_sh, c_noise,
           W_attr, W_sh, W_out, w_scale):
    out_dtype = jnp.result_type(features.dtype, W_attr.dtype, W_out.dtype)
    eidx = edge_index.reshape(-1).astype(jnp.int32)
    coef = _compute_coef(edge_attr.T, edge_sh.T,
                         W_attr.astype(jnp.float32),
                         W_sh.astype(jnp.float32))
    partials = _sc_scatter(features, eidx, coef)
    new_features = _epilogue(
        partials, W_out.astype(jnp.float32),
        c_noise.reshape(_N, 1).astype(jnp.float32),
        w_scale.reshape(1, _D).astype(jnp.float32))
    return (coords, new_features.astype(out_dtype))


# speedup vs baseline: 2.8183x; 1.9069x over previous
import functools
import math

import jax
import jax.numpy as jnp
from jax import lax
from jax.experimental import pallas as pl
from jax.experimental.pallas import tpu as pltpu
from jax.experimental.pallas import tpu_sc as plsc

_N = 10000
_E = 320000
_D = 128
_SH = 9
_EA = 16
_INV_SQRT_DEG = 1.0 / math.sqrt(32.0)

_NC = 2
_NS = 16
_NW = _NC * _NS
_EPW = _E // _NW
_G = 80
_NCHUNK = _EPW // _G
_NP = 10240
_RPS = _NP // _NS
_ZR = 80

_BE = 6400
_BN = 2000


def _i0(i):
    return i * 0


def _coef_body(eat_ref, sht_ref, wa_ref, ws_ref, o_ref):
    g = lax.dot_general(
        eat_ref[...], wa_ref[...], (((0,), (0,)), ((), ())),
        preferred_element_type=jnp.float32)
    s = lax.dot_general(
        sht_ref[...], ws_ref[...], (((0,), (0,)), ((), ())),
        preferred_element_type=jnp.float32)
    o_ref[...] = jax.nn.sigmoid(g) * s


def _compute_coef(eaT, shT, Wa, Ws):
    return pl.pallas_call(
        _coef_body,
        grid=(_E // _BE,),
        in_specs=[
            pl.BlockSpec((_EA, _BE), lambda i: (_i0(i), i)),
            pl.BlockSpec((_SH, _BE), lambda i: (_i0(i), i)),
            pl.BlockSpec((_EA, _D), lambda i: (_i0(i), _i0(i))),
            pl.BlockSpec((_SH, _D), lambda i: (_i0(i), _i0(i))),
        ],
        out_specs=pl.BlockSpec((_BE, _D), lambda i: (i, _i0(i))),
        out_shape=jax.ShapeDtypeStruct((_E, _D), jnp.float32),
    )(eaT, shT, Wa, Ws)


def _loop32(lo, hi, body_fn):
    lax.fori_loop(jnp.int32(lo), jnp.int32(hi),
                  lambda i, c: (body_fn(i), c)[1], None)


_NDAT = 2
_NIDX = 4


def _sc_body(feat_hbm, eidx_hbm, coef_hbm, out_hbm,
             sidx0, sidx1, sidx2, sidx3, didx0, didx1, didx2, didx3,
             rows0, rows1, coefb0, coefb1, agg_sh,
             si0, si1, si2, si3, sg0, sg1, sc0, sc1):
    cid = lax.axis_index("c").astype(jnp.int32)
    sid = lax.axis_index("s").astype(jnp.int32)
    wid = cid * jnp.int32(_NS) + sid
    base = wid * jnp.int32(_EPW)

    sidx = (sidx0, sidx1, sidx2, sidx3)
    didx = (didx0, didx1, didx2, didx3)
    rows = (rows0, rows1)
    coefb = (coefb0, coefb1)
    si = (si0, si1, si2, si3)
    sg = (sg0, sg1)
    sc = (sc0, sc1)

    def _issue_idx(k, q):
        off = base + k * jnp.int32(_G)
        pltpu.async_copy(eidx_hbm.at[pl.ds(off, _G)], sidx[q], si[q])
        pltpu.async_copy(
            eidx_hbm.at[pl.ds(off + jnp.int32(_E), _G)], didx[q], si[q])

    def _wait_idx(q):
        pltpu.make_async_copy(eidx_hbm.at[pl.ds(0, _G)], sidx[q], si[q]).wait()
        pltpu.make_async_copy(eidx_hbm.at[pl.ds(0, _G)], didx[q], si[q]).wait()

    def _issue_dat(k, b, q):
        pltpu.async_copy(feat_hbm.at[sidx[q]], rows[b], sg[b])
        off = base + k * jnp.int32(_G)
        pltpu.async_copy(coef_hbm.at[pl.ds(off, _G)], coefb[b], sc[b])

    def _wait_dat(b, q):
        pltpu.make_async_copy(feat_hbm.at[sidx[q]], rows[b], sg[b]).wait()
        pltpu.make_async_copy(
            coef_hbm.at[pl.ds(0, _G)], coefb[b], sc[b]).wait()

    for q in range(_NIDX):
        _issue_idx(jnp.int32(q), q)

    def _zero_stage(i):
        def _zs(j16):
            rows0.at[i, pl.ds(j16 * jnp.int32(16), 16)][...] = (
                jnp.zeros((16,), jnp.float32))
        _loop32(0, _D // 16, _zs)
    _loop32(0, _ZR, _zero_stage)

    for r in range(0, _RPS, _ZR):
        pltpu.sync_copy(rows0, agg_sh.at[pl.ds(sid * jnp.int32(_RPS) + jnp.int32(r), _ZR)])
    plsc.subcore_barrier()

    _wait_idx(0)
    _issue_dat(jnp.int32(0), 0, 0)
    _wait_idx(1)
    _issue_dat(jnp.int32(1), 1, 1)

    def _mul(b):
        def _row(i):
            for j in range(_D // 16):
                sl = pl.ds(jnp.int32(16 * j), 16)
                rows[b].at[i, sl][...] = (
                    rows[b].at[i, sl][...] * coefb[b].at[i, sl][...])
        _loop32(0, _G, _row)

    def _step(k, b, q):
        _wait_dat(b, q)
        _mul(b)
        pltpu.sync_copy(rows[b], agg_sh.at[didx[q]], add=True)

        @pl.when(k + jnp.int32(2) < jnp.int32(_NCHUNK))
        def _():
            q2 = (q + 2) % _NIDX
            _wait_idx(q2)
            _issue_dat(k + jnp.int32(2), b, q2)

        @pl.when(k + jnp.int32(_NIDX) < jnp.int32(_NCHUNK))
        def _():
            _issue_idx(k + jnp.int32(_NIDX), q)

    def _group(t):
        k0 = t * jnp.int32(_NDAT * _NIDX)
        for j in range(_NDAT * _NIDX):
            kj = k0 + jnp.int32(j)

            @pl.when(kj < jnp.int32(_NCHUNK))
            def _():
                _step(kj, j % _NDAT, j % _NIDX)
    _loop32(0, (_NCHUNK + _NDAT * _NIDX - 1) // (_NDAT * _NIDX), _group)

    plsc.subcore_barrier()

    for r in range(0, _RPS, _ZR):
        row0 = sid * jnp.int32(_RPS) + jnp.int32(r)
        pltpu.sync_copy(agg_sh.at[pl.ds(row0, _ZR)], rows0)
        pltpu.sync_copy(rows0, out_hbm.at[cid, pl.ds(row0, _ZR)])


def _sc_scatter(features, eidx4, coef):
    mesh = plsc.VectorSubcoreMesh(core_axis_name="c", subcore_axis_name="s")
    k = pl.kernel(
        _sc_body,
        mesh=mesh,
        out_type=jax.ShapeDtypeStruct((_NC, _NP, _D), jnp.float32),
        scratch_types=(
            [pltpu.VMEM((_G,), jnp.int32)] * (2 * _NIDX)
            + [pltpu.VMEM((_G, _D), jnp.float32)] * (2 * _NDAT)
            + [pltpu.VMEM_SHARED((_NP, _D), jnp.float32)]
            + [pltpu.SemaphoreType.DMA] * (_NIDX + 2 * _NDAT)
        ),
    )
    return k(features, eidx4, coef)


def _out_body(agg_ref, wout_ref, cn_ref, ws_ref, o_ref):
    agg = (agg_ref[0] + agg_ref[1]) * _INV_SQRT_DEG
    out = lax.dot_general(
        agg, wout_ref[...], (((1,), (0,)), ((), ())),
        precision=lax.Precision.HIGHEST, preferred_element_type=jnp.float32)
    scale = 1.0 + jnp.tanh(cn_ref[...]) * ws_ref[...]
    o_ref[...] = out * scale


def _epilogue(partials, W_out, c_noise, w_scale):
    return pl.pallas_call(
        _out_body,
        grid=(_N // _BN,),
        in_specs=[
            pl.BlockSpec((_NC, _BN, _D), lambda i: (_i0(i), i, _i0(i))),
            pl.BlockSpec((_D, _D), lambda i: (_i0(i), _i0(i))),
            pl.BlockSpec((_BN, 1), lambda i: (i, _i0(i))),
            pl.BlockSpec((1, _D), lambda i: (_i0(i), _i0(i))),
        ],
        out_specs=pl.BlockSpec((_BN, _D), lambda i: (i, _i0(i))),
        out_shape=jax.ShapeDtypeStruct((_N, _D), jnp.float32),
    )(partials, W_out, c_noise, w_scale)


def kernel(features, coords, edge_index, edge_attr, edge_sh, c_noise,
           W_attr, W_sh, W_out, w_scale):
    out_dtype = jnp.result_type(features.dtype, W_attr.dtype, W_out.dtype)
    eidx4 = edge_index.reshape(-1).astype(jnp.int32)
    coef = _compute_coef(edge_attr.T, edge_sh.T,
                         W_attr.astype(jnp.float32),
                         W_sh.astype(jnp.float32))
    partials = _sc_scatter(features, eidx4, coef)
    new_features = _epilogue(
        partials, W_out.astype(jnp.float32),
        c_noise.reshape(_N, 1).astype(jnp.float32),
        w_scale.reshape(1, _D).astype(jnp.float32))
    return (coords, new_features.astype(out_dtype))


# speedup vs baseline: 2.9328x; 1.0406x over previous
import functools
import math

import jax
import jax.numpy as jnp
from jax import lax
from jax.experimental import pallas as pl
from jax.experimental.pallas import tpu as pltpu
from jax.experimental.pallas import tpu_sc as plsc

_N = 10000
_E = 320000
_D = 128
_SH = 9
_EA = 16
_INV_SQRT_DEG = 1.0 / math.sqrt(32.0)

_NC = 2
_NS = 16
_NW = _NC * _NS
_EPW = _E // _NW
_G = 80
_NCHUNK = _EPW // _G
_NP = 10240
_RPS = _NP // _NS
_ZR = 80

_BE = 6400
_BN = 2000


def _i0(i):
    return i * 0


def _coef_body(eat_ref, sht_ref, w_ref, o_ref):
    x = jnp.concatenate([eat_ref[...], sht_ref[...]], axis=0)
    y = lax.dot_general(
        x, w_ref[...], (((0,), (0,)), ((), ())),
        preferred_element_type=jnp.float32)
    o_ref[...] = jax.nn.sigmoid(y[:, :_D]) * y[:, _D:]


def _compute_coef(eaT, shT, W_cat):
    return pl.pallas_call(
        _coef_body,
        grid=(_E // _BE,),
        in_specs=[
            pl.BlockSpec((_EA, _BE), lambda i: (_i0(i), i)),
            pl.BlockSpec((_SH, _BE), lambda i: (_i0(i), i)),
            pl.BlockSpec((_EA + _SH, 2 * _D), lambda i: (_i0(i), _i0(i))),
        ],
        out_specs=pl.BlockSpec((_BE, _D), lambda i: (i, _i0(i))),
        out_shape=jax.ShapeDtypeStruct((_E, _D), jnp.float32),
    )(eaT, shT, W_cat)


def _loop32(lo, hi, body_fn):
    lax.fori_loop(jnp.int32(lo), jnp.int32(hi),
                  lambda i, c: (body_fn(i), c)[1], None)


_NDAT = 2
_NIDX = 4


def _sc_body(feat_hbm, eidx_hbm, coef_hbm, out_hbm,
             sidx0, sidx1, sidx2, sidx3, didx0, didx1, didx2, didx3,
             rows0, rows1, coefb0, coefb1, agg_sh,
             si0, si1, si2, si3, sg0, sg1, sc0, sc1):
    cid = lax.axis_index("c").astype(jnp.int32)
    sid = lax.axis_index("s").astype(jnp.int32)
    wid = cid * jnp.int32(_NS) + sid
    base = wid * jnp.int32(_EPW)

    sidx = (sidx0, sidx1, sidx2, sidx3)
    didx = (didx0, didx1, didx2, didx3)
    rows = (rows0, rows1)
    coefb = (coefb0, coefb1)
    si = (si0, si1, si2, si3)
    sg = (sg0, sg1)
    sc = (sc0, sc1)

    def _issue_idx(k, q):
        off = base + k * jnp.int32(_G)
        pltpu.async_copy(eidx_hbm.at[pl.ds(off, _G)], sidx[q], si[q])
        pltpu.async_copy(
            eidx_hbm.at[pl.ds(off + jnp.int32(_E), _G)], didx[q], si[q])

    def _wait_idx(q):
        pltpu.make_async_copy(eidx_hbm.at[pl.ds(0, _G)], sidx[q], si[q]).wait()
        pltpu.make_async_copy(eidx_hbm.at[pl.ds(0, _G)], didx[q], si[q]).wait()

    def _issue_dat(k, b, q):
        pltpu.async_copy(feat_hbm.at[sidx[q]], rows[b], sg[b])
        off = base + k * jnp.int32(_G)
        pltpu.async_copy(coef_hbm.at[pl.ds(off, _G)], coefb[b], sc[b])

    def _wait_dat(b, q):
        pltpu.make_async_copy(feat_hbm.at[sidx[q]], rows[b], sg[b]).wait()
        pltpu.make_async_copy(
            coef_hbm.at[pl.ds(0, _G)], coefb[b], sc[b]).wait()

    for q in range(_NIDX):
        _issue_idx(jnp.int32(q), q)

    def _zero_stage(i):
        def _zs(j16):
            rows0.at[i, pl.ds(j16 * jnp.int32(16), 16)][...] = (
                jnp.zeros((16,), jnp.float32))
        _loop32(0, _D // 16, _zs)
    _loop32(0, _ZR, _zero_stage)

    for r in range(0, _RPS, _ZR):
        pltpu.sync_copy(rows0, agg_sh.at[pl.ds(sid * jnp.int32(_RPS) + jnp.int32(r), _ZR)])
    plsc.subcore_barrier()

    _wait_idx(0)
    _issue_dat(jnp.int32(0), 0, 0)
    _wait_idx(1)
    _issue_dat(jnp.int32(1), 1, 1)

    def _mul(b):
        def _row(i):
            for j in range(_D // 16):
                sl = pl.ds(jnp.int32(16 * j), 16)
                rows[b].at[i, sl][...] = (
                    rows[b].at[i, sl][...] * coefb[b].at[i, sl][...])
        _loop32(0, _G, _row)

    def _step(k, b, q):
        _wait_dat(b, q)
        _mul(b)
        pltpu.sync_copy(rows[b], agg_sh.at[didx[q]], add=True)

        @pl.when(k + jnp.int32(2) < jnp.int32(_NCHUNK))
        def _():
            q2 = (q + 2) % _NIDX
            _wait_idx(q2)
            _issue_dat(k + jnp.int32(2), b, q2)

        @pl.when(k + jnp.int32(_NIDX) < jnp.int32(_NCHUNK))
        def _():
            _issue_idx(k + jnp.int32(_NIDX), q)

    def _group(t):
        k0 = t * jnp.int32(_NDAT * _NIDX)
        for j in range(_NDAT * _NIDX):
            kj = k0 + jnp.int32(j)

            @pl.when(kj < jnp.int32(_NCHUNK))
            def _():
                _step(kj, j % _NDAT, j % _NIDX)
    _loop32(0, (_NCHUNK + _NDAT * _NIDX - 1) // (_NDAT * _NIDX), _group)

    plsc.subcore_barrier()

    for r in range(0, _RPS, _ZR):
        row0 = sid * jnp.int32(_RPS) + jnp.int32(r)
        pltpu.sync_copy(agg_sh.at[pl.ds(row0, _ZR)], rows0)
        pltpu.sync_copy(rows0, out_hbm.at[cid, pl.ds(row0, _ZR)])


def _sc_scatter(features, eidx4, coef):
    mesh = plsc.VectorSubcoreMesh(core_axis_name="c", subcore_axis_name="s")
    k = pl.kernel(
        _sc_body,
        mesh=mesh,
        out_type=jax.ShapeDtypeStruct((_NC, _NP, _D), jnp.float32),
        scratch_types=(
            [pltpu.VMEM((_G,), jnp.int32)] * (2 * _NIDX)
            + [pltpu.VMEM((_G, _D), jnp.float32)] * (2 * _NDAT)
            + [pltpu.VMEM_SHARED((_NP, _D), jnp.float32)]
            + [pltpu.SemaphoreType.DMA] * (_NIDX + 2 * _NDAT)
        ),
    )
    return k(features, eidx4, coef)


def _out_body(agg_ref, wout_ref, cn_ref, ws_ref, o_ref):
    agg = (agg_ref[0] + agg_ref[1]) * _INV_SQRT_DEG
    out = lax.dot_general(
        agg, wout_ref[...], (((1,), (0,)), ((), ())),
        precision=lax.Precision.HIGHEST, preferred_element_type=jnp.float32)
    scale = 1.0 + jnp.tanh(cn_ref[...]) * ws_ref[...]
    o_ref[...] = out * scale


def _epilogue(partials, W_out, c_noise, w_scale):
    return pl.pallas_call(
        _out_body,
        grid=(_N // _BN,),
        in_specs=[
            pl.BlockSpec((_NC, _BN, _D), lambda i: (_i0(i), i, _i0(i))),
            pl.BlockSpec((_D, _D), lambda i: (_i0(i), _i0(i))),
            pl.BlockSpec((_BN, 1), lambda i: (i, _i0(i))),
            pl.BlockSpec((1, _D), lambda i: (_i0(i), _i0(i))),
        ],
        out_specs=pl.BlockSpec((_BN, _D), lambda i: (i, _i0(i))),
        out_shape=jax.ShapeDtypeStruct((_N, _D), jnp.float32),
    )(partials, W_out, c_noise, w_scale)


def kernel(features, coords, edge_index, edge_attr, edge_sh, c_noise,
           W_attr, W_sh, W_out, w_scale):
    out_dtype = jnp.result_type(features.dtype, W_attr.dtype, W_out.dtype)
    eidx4 = edge_index.reshape(-1).astype(jnp.int32)
    W_cat = jnp.zeros((_EA + _SH, 2 * _D), jnp.float32)
    W_cat = W_cat.at[:_EA, :_D].set(W_attr.astype(jnp.float32))
    W_cat = W_cat.at[_EA:, _D:].set(W_sh.astype(jnp.float32))
    coef = _compute_coef(edge_attr.T, edge_sh.T, W_cat)
    partials = _sc_scatter(features, eidx4, coef)
    new_features = _epilogue(
        partials, W_out.astype(jnp.float32),
        c_noise.reshape(_N, 1).astype(jnp.float32),
        w_scale.reshape(1, _D).astype(jnp.float32))
    return (coords, new_features.astype(out_dtype))


# speedup vs baseline: 3.0934x; 1.0548x over previous
import functools
import math

import jax
import jax.numpy as jnp
from jax import lax
from jax.experimental import pallas as pl
from jax.experimental.pallas import tpu as pltpu
from jax.experimental.pallas import tpu_sc as plsc

_N = 10000
_E = 320000
_D = 128
_SH = 9
_EA = 16
_INV_SQRT_DEG = 1.0 / math.sqrt(32.0)

_NC = 2
_NS = 16
_NW = _NC * _NS
_EPW = _E // _NW
_G = 80
_NCHUNK = _EPW // _G
_NP = 10240
_RPS = _NP // _NS
_ZR = 80

_BE = 16000
_BN = 2000


def _i0(i):
    return i * 0


def _coef_body(eat_ref, sht_ref, w_ref, o_ref):
    x = jnp.concatenate([eat_ref[...], sht_ref[...]], axis=0)
    y = lax.dot_general(
        x, w_ref[...], (((0,), (0,)), ((), ())),
        preferred_element_type=jnp.float32)
    o_ref[...] = jax.nn.sigmoid(y[:, :_D]) * y[:, _D:]


def _compute_coef(eaT, shT, W_cat):
    return pl.pallas_call(
        _coef_body,
        grid=(_E // _BE,),
        in_specs=[
            pl.BlockSpec((_EA, _BE), lambda i: (_i0(i), i)),
            pl.BlockSpec((_SH, _BE), lambda i: (_i0(i), i)),
            pl.BlockSpec((_EA + _SH, 2 * _D), lambda i: (_i0(i), _i0(i))),
        ],
        out_specs=pl.BlockSpec((_BE, _D), lambda i: (i, _i0(i))),
        out_shape=jax.ShapeDtypeStruct((_E, _D), jnp.float32),
    )(eaT, shT, W_cat)


def _loop32(lo, hi, body_fn):
    lax.fori_loop(jnp.int32(lo), jnp.int32(hi),
                  lambda i, c: (body_fn(i), c)[1], None)


_NDAT = 2
_NIDX = 4


def _sc_body(feat_hbm, eidx_hbm, coef_hbm, out_hbm,
             sidx0, sidx1, sidx2, sidx3, didx0, didx1, didx2, didx3,
             rows0, rows1, coefb0, coefb1, agg_sh,
             si0, si1, si2, si3, sg0, sg1, sc0, sc1):
    cid = lax.axis_index("c").astype(jnp.int32)
    sid = lax.axis_index("s").astype(jnp.int32)
    wid = cid * jnp.int32(_NS) + sid
    base = wid * jnp.int32(_EPW)

    sidx = (sidx0, sidx1, sidx2, sidx3)
    didx = (didx0, didx1, didx2, didx3)
    rows = (rows0, rows1)
    coefb = (coefb0, coefb1)
    si = (si0, si1, si2, si3)
    sg = (sg0, sg1)
    sc = (sc0, sc1)

    def _issue_idx(k, q):
        off = base + k * jnp.int32(_G)
        pltpu.async_copy(eidx_hbm.at[pl.ds(off, _G)], sidx[q], si[q])
        pltpu.async_copy(
            eidx_hbm.at[pl.ds(off + jnp.int32(_E), _G)], didx[q], si[q])

    def _wait_idx(q):
        pltpu.make_async_copy(eidx_hbm.at[pl.ds(0, _G)], sidx[q], si[q]).wait()
        pltpu.make_async_copy(eidx_hbm.at[pl.ds(0, _G)], didx[q], si[q]).wait()

    def _issue_dat(k, b, q):
        pltpu.async_copy(feat_hbm.at[sidx[q]], rows[b], sg[b])
        off = base + k * jnp.int32(_G)
        pltpu.async_copy(coef_hbm.at[pl.ds(off, _G)], coefb[b], sc[b])

    def _wait_dat(b, q):
        pltpu.make_async_copy(feat_hbm.at[sidx[q]], rows[b], sg[b]).wait()
        pltpu.make_async_copy(
            coef_hbm.at[pl.ds(0, _G)], coefb[b], sc[b]).wait()

    for q in range(_NIDX):
        _issue_idx(jnp.int32(q), q)

    def _zero_stage(i):
        def _zs(j16):
            rows0.at[i, pl.ds(j16 * jnp.int32(16), 16)][...] = (
                jnp.zeros((16,), jnp.float32))
        _loop32(0, _D // 16, _zs)
    _loop32(0, _ZR, _zero_stage)

    for r in range(0, _RPS, _ZR):
        pltpu.sync_copy(rows0, agg_sh.at[pl.ds(sid * jnp.int32(_RPS) + jnp.int32(r), _ZR)])
    plsc.subcore_barrier()

    _wait_idx(0)
    _issue_dat(jnp.int32(0), 0, 0)
    _wait_idx(1)
    _issue_dat(jnp.int32(1), 1, 1)

    def _mul(b):
        def _row(i):
            for j in range(_D // 16):
                sl = pl.ds(jnp.int32(16 * j), 16)
                rows[b].at[i, sl][...] = (
                    rows[b].at[i, sl][...] * coefb[b].at[i, sl][...])
        _loop32(0, _G, _row)

    def _step(k, b, q):
        _wait_dat(b, q)
        _mul(b)
        pltpu.sync_copy(rows[b], agg_sh.at[didx[q]], add=True)

        @pl.when(k + jnp.int32(2) < jnp.int32(_NCHUNK))
        def _():
            q2 = (q + 2) % _NIDX
            _wait_idx(q2)
            _issue_dat(k + jnp.int32(2), b, q2)

        @pl.when(k + jnp.int32(_NIDX) < jnp.int32(_NCHUNK))
        def _():
            _issue_idx(k + jnp.int32(_NIDX), q)

    def _group(t):
        k0 = t * jnp.int32(_NDAT * _NIDX)
        for j in range(_NDAT * _NIDX):
            kj = k0 + jnp.int32(j)

            @pl.when(kj < jnp.int32(_NCHUNK))
            def _():
                _step(kj, j % _NDAT, j % _NIDX)
    _loop32(0, (_NCHUNK + _NDAT * _NIDX - 1) // (_NDAT * _NIDX), _group)

    plsc.subcore_barrier()

    for r in range(0, _RPS, _ZR):
        row0 = sid * jnp.int32(_RPS) + jnp.int32(r)
        pltpu.sync_copy(agg_sh.at[pl.ds(row0, _ZR)], rows0)
        pltpu.sync_copy(rows0, out_hbm.at[cid, pl.ds(row0, _ZR)])


def _sc_scatter(features, eidx4, coef):
    mesh = plsc.VectorSubcoreMesh(core_axis_name="c", subcore_axis_name="s")
    k = pl.kernel(
        _sc_body,
        mesh=mesh,
        out_type=jax.ShapeDtypeStruct((_NC, _NP, _D), jnp.float32),
        scratch_types=(
            [pltpu.VMEM((_G,), jnp.int32)] * (2 * _NIDX)
            + [pltpu.VMEM((_G, _D), jnp.float32)] * (2 * _NDAT)
            + [pltpu.VMEM_SHARED((_NP, _D), jnp.float32)]
            + [pltpu.SemaphoreType.DMA] * (_NIDX + 2 * _NDAT)
        ),
    )
    return k(features, eidx4, coef)


def _out_body(agg_ref, wout_ref, cn_ref, ws_ref, o_ref):
    agg = (agg_ref[0] + agg_ref[1]) * _INV_SQRT_DEG
    out = lax.dot_general(
        agg, wout_ref[...], (((1,), (0,)), ((), ())),
        precision=lax.Precision.HIGHEST, preferred_element_type=jnp.float32)
    scale = 1.0 + jnp.tanh(cn_ref[...]) * ws_ref[...]
    o_ref[...] = out * scale


def _epilogue(partials, W_out, c_noise, w_scale):
    return pl.pallas_call(
        _out_body,
        grid=(_N // _BN,),
        in_specs=[
            pl.BlockSpec((_NC, _BN, _D), lambda i: (_i0(i), i, _i0(i))),
            pl.BlockSpec((_D, _D), lambda i: (_i0(i), _i0(i))),
            pl.BlockSpec((_BN, 1), lambda i: (i, _i0(i))),
            pl.BlockSpec((1, _D), lambda i: (_i0(i), _i0(i))),
        ],
        out_specs=pl.BlockSpec((_BN, _D), lambda i: (i, _i0(i))),
        out_shape=jax.ShapeDtypeStruct((_N, _D), jnp.float32),
    )(partials, W_out, c_noise, w_scale)


def kernel(features, coords, edge_index, edge_attr, edge_sh, c_noise,
           W_attr, W_sh, W_out, w_scale):
    out_dtype = jnp.result_type(features.dtype, W_attr.dtype, W_out.dtype)
    eidx4 = edge_index.reshape(-1).astype(jnp.int32)
    W_cat = jnp.zeros((_EA + _SH, 2 * _D), jnp.float32)
    W_cat = W_cat.at[:_EA, :_D].set(W_attr.astype(jnp.float32))
    W_cat = W_cat.at[_EA:, _D:].set(W_sh.astype(jnp.float32))
    coef = _compute_coef(edge_attr.T, edge_sh.T, W_cat)
    partials = _sc_scatter(features, eidx4, coef)
    new_features = _epilogue(
        partials, W_out.astype(jnp.float32),
        c_noise.reshape(_N, 1).astype(jnp.float32),
        w_scale.reshape(1, _D).astype(jnp.float32))
    return (coords, new_features.astype(out_dtype))
